# baseline (device time: 62326 ns/iter reference)
import jax
import jax.numpy as jnp
from jax import lax
from jax.experimental import pallas as pl
from jax.experimental.pallas import tpu as pltpu

N_DEV = 16
N_STAGES = 4
SCALE = 0.08838834764831843

R = 2048
RL = 16
RT = R + RL


def kernel(x, Wq, Wo, K_ext, V_ext):
    x2 = x.reshape(256, 1024)
    K2 = K_ext.reshape(4096, 256)
    V2 = V_ext.reshape(4096, 256)

    def body(x_ref, wq_ref, wo_ref, k_ref, v_ref, out_ref,
             acc_ref, send_ref, recv_ref, send_sems, recv_sems):
        my = lax.axis_index("i")

        barrier_sem = pltpu.get_barrier_semaphore()
        for s in range(N_STAGES):
            pl.semaphore_signal(
                barrier_sem, inc=1,
                device_id=(my ^ (1 << s),),
                device_id_type=pl.DeviceIdType.MESH,
            )
        pl.semaphore_wait(barrier_sem, N_STAGES)

        xq = x_ref[...].astype(jnp.bfloat16)
        wq = wq_ref[...].astype(jnp.bfloat16)
        q = jnp.dot(xq, wq, preferred_element_type=jnp.float32)
        q = (q * SCALE).astype(jnp.bfloat16)

        ri = lax.broadcasted_iota(jnp.int32, (1024, 128), 0)
        ji = lax.broadcasted_iota(jnp.int32, (1024, 128), 1)
        M1024 = (ri % 128 == ji).astype(jnp.float32)
        ai = lax.broadcasted_iota(jnp.int32, (8, 1024), 0)
        ar = lax.broadcasted_iota(jnp.int32, (8, 1024), 1)
        A8 = (ar // 128 == ai).astype(jnp.float32)

        ones_cols = jnp.ones((4096, 128), jnp.bfloat16)
        for g in range(2):
            kg = k_ref[:, g * 128:(g + 1) * 128].astype(jnp.bfloat16)
            vg = v_ref[:, g * 128:(g + 1) * 128].astype(jnp.bfloat16)
            vext = jnp.concatenate([vg, ones_cols], axis=1)
            q4 = jnp.concatenate(
                [q[:, (4 * g + j) * 128:(4 * g + j + 1) * 128]
                 for j in range(4)], axis=0)
            s = lax.dot_general(
                q4, kg, (((1,), (1,)), ((), ())),
                preferred_element_type=jnp.float32)
            p = jnp.exp(s)
            o_ext = jnp.dot(p.astype(jnp.bfloat16), vext,
                            preferred_element_type=jnp.float32)
            o_g = o_ext[:, :128]
            l_rep = o_ext[:, 128:]
            m_g = jnp.dot(A8, l_rep * M1024,
                          preferred_element_type=jnp.float32)
            acc_ref[g * 1024:(g + 1) * 1024, :] = o_g
            acc_ref[R + g * 8:R + (g + 1) * 8, :] = m_g
            send_ref[g * 1024:(g + 1) * 1024, :] = o_g.astype(jnp.bfloat16)
            send_ref[R + g * 8:R + (g + 1) * 8, :] = m_g.astype(jnp.bfloat16)

        for stage in range(N_STAGES):
            partner = my ^ (1 << stage)
            rdma = pltpu.make_async_remote_copy(
                src_ref=send_ref,
                dst_ref=recv_ref.at[stage],
                send_sem=send_sems.at[stage],
                recv_sem=recv_sems.at[stage],
                device_id=(partner,),
                device_id_type=pl.DeviceIdType.MESH,
            )
            rdma.start()
            rdma.wait()
            new = acc_ref[...] + recv_ref[stage].astype(jnp.float32)
            acc_ref[...] = new
            if stage < N_STAGES - 1:
                send_ref[...] = new.astype(jnp.bfloat16)

        a = acc_ref[...]
        ei = lax.broadcasted_iota(jnp.int32, (R, RL), 1)
        er = lax.broadcasted_iota(jnp.int32, (R, RL), 0)
        E = (er // 128 == ei).astype(jnp.float32)
        l_rows = jnp.dot(E, a[R:RT, :],
                         preferred_element_type=jnp.float32)
        ri2 = lax.broadcasted_iota(jnp.int32, (R, 128), 0)
        ji2 = lax.broadcasted_iota(jnp.int32, (R, 128), 1)
        Mfull = (ri2 % 128 == ji2).astype(jnp.float32)
        l_col = jnp.sum(l_rows * Mfull, axis=1, keepdims=True)
        on = (a[:R, :] / l_col).astype(jnp.bfloat16)
        out = jnp.zeros((256, 1024), jnp.float32)
        for h in range(8):
            woh = wo_ref[h * 128:(h + 1) * 128, :].astype(jnp.bfloat16)
            out = out + jnp.dot(on[h * 256:(h + 1) * 256, :], woh,
                                preferred_element_type=jnp.float32)
        out_ref[...] = out

    out2 = pl.pallas_call(
        body,
        out_shape=jax.ShapeDtypeStruct((256, 1024), jnp.float32),
        in_specs=[pl.BlockSpec(memory_space=pltpu.VMEM)] * 5,
        out_specs=pl.BlockSpec(memory_space=pltpu.VMEM),
        scratch_shapes=[
            pltpu.VMEM((RT, 128), jnp.float32),
            pltpu.VMEM((RT, 128), jnp.bfloat16),
            pltpu.VMEM((N_STAGES, RT, 128), jnp.bfloat16),
            pltpu.SemaphoreType.DMA((N_STAGES,)),
            pltpu.SemaphoreType.DMA((N_STAGES,)),
        ],
        compiler_params=pltpu.CompilerParams(collective_id=0),
    )(x2, Wq, Wo, K2, V2)

    return out2.reshape(1, 256, 1024)


# device time: 48514 ns/iter; 1.2847x vs baseline; 1.2847x over previous
import jax
import jax.numpy as jnp
from jax import lax
from jax.experimental import pallas as pl
from jax.experimental.pallas import tpu as pltpu

N_DEV = 16
N_STAGES = 4
SCALE = 0.08838834764831843

R = 1024
RL = 8
RT = R + RL

ORDER = ((1, 2, 4, 8), (8, 4, 2, 1))


def kernel(x, Wq, Wo, K_ext, V_ext):
    x2 = x.reshape(256, 1024)
    K2 = K_ext.reshape(4096, 256)
    V2 = V_ext.reshape(4096, 256)

    def body(x_ref, wq_ref, wo_ref, k_ref, v_ref, out_ref,
             acc_ref, send_ref, recv_ref, send_sems, recv_sems):
        my = lax.axis_index("i")

        barrier_sem = pltpu.get_barrier_semaphore()
        for d in (1, 2, 4, 8):
            pl.semaphore_signal(
                barrier_sem, inc=1,
                device_id=(my ^ d,),
                device_id_type=pl.DeviceIdType.MESH,
            )
        pl.semaphore_wait(barrier_sem, N_STAGES)

        xq = x_ref[...].astype(jnp.bfloat16)
        wq = wq_ref[...].astype(jnp.bfloat16)
        q = jnp.dot(xq, wq, preferred_element_type=jnp.float32)
        q = (q * SCALE).astype(jnp.bfloat16)

        ri = lax.broadcasted_iota(jnp.int32, (R, 128), 0)
        ji = lax.broadcasted_iota(jnp.int32, (R, 128), 1)
        M1024 = (ri % 128 == ji).astype(jnp.float32)
        ai = lax.broadcasted_iota(jnp.int32, (RL, R), 0)
        ar = lax.broadcasted_iota(jnp.int32, (RL, R), 1)
        A8 = (ar // 128 == ai).astype(jnp.float32)

        ones_cols = jnp.ones((4096, 128), jnp.bfloat16)

        def compute_flow(g):
            kg = k_ref[:, g * 128:(g + 1) * 128].astype(jnp.bfloat16)
            vg = v_ref[:, g * 128:(g + 1) * 128].astype(jnp.bfloat16)
            vext = jnp.concatenate([vg, ones_cols], axis=1)
            q4 = jnp.concatenate(
                [q[:, (4 * g + j) * 128:(4 * g + j + 1) * 128]
                 for j in range(4)], axis=0)
            s = lax.dot_general(
                q4, kg, (((1,), (1,)), ((), ())),
                preferred_element_type=jnp.float32)
            p = jnp.exp(s)
            o_ext = jnp.dot(p.astype(jnp.bfloat16), vext,
                            preferred_element_type=jnp.float32)
            o_g = o_ext[:, :128]
            l_rep = o_ext[:, 128:]
            m_g = jnp.dot(A8, l_rep * M1024,
                          preferred_element_type=jnp.float32)
            acc_ref[g, :R, :] = o_g
            acc_ref[g, R:, :] = m_g
            send_ref[g, :R, :] = o_g.astype(jnp.bfloat16)
            send_ref[g, R:, :] = m_g.astype(jnp.bfloat16)

        def start_stage(g, stage):
            partner = my ^ ORDER[g][stage]
            rdma = pltpu.make_async_remote_copy(
                src_ref=send_ref.at[g],
                dst_ref=recv_ref.at[g, stage],
                send_sem=send_sems.at[g, stage],
                recv_sem=recv_sems.at[g, stage],
                device_id=(partner,),
                device_id_type=pl.DeviceIdType.MESH,
            )
            rdma.start()
            return rdma

        rd = {}
        compute_flow(0)
        rd[(0, 0)] = start_stage(0, 0)
        compute_flow(1)
        rd[(1, 0)] = start_stage(1, 0)

        for stage in range(N_STAGES):
            for g in range(2):
                rd[(g, stage)].wait()
                new = acc_ref[g] + recv_ref[g, stage].astype(jnp.float32)
                acc_ref[g] = new
                if stage < N_STAGES - 1:
                    send_ref[g] = new.astype(jnp.bfloat16)
                    rd[(g, stage + 1)] = start_stage(g, stage + 1)

        ei = lax.broadcasted_iota(jnp.int32, (R, RL), 1)
        er = lax.broadcasted_iota(jnp.int32, (R, RL), 0)
        E = (er // 128 == ei).astype(jnp.float32)
        out = jnp.zeros((256, 1024), jnp.float32)
        for g in range(2):
            a = acc_ref[g]
            l_rows = jnp.dot(E, a[R:, :],
                             preferred_element_type=jnp.float32)
            l_col = jnp.sum(l_rows * M1024, axis=1, keepdims=True)
            on = (a[:R, :] / l_col).astype(jnp.bfloat16)
            for j in range(4):
                h = 4 * g + j
                woh = wo_ref[h * 128:(h + 1) * 128, :].astype(jnp.bfloat16)
                out = out + jnp.dot(on[j * 256:(j + 1) * 256, :], woh,
                                    preferred_element_type=jnp.float32)
        out_ref[...] = out

    out2 = pl.pallas_call(
        body,
        out_shape=jax.ShapeDtypeStruct((256, 1024), jnp.float32),
        in_specs=[pl.BlockSpec(memory_space=pltpu.VMEM)] * 5,
        out_specs=pl.BlockSpec(memory_space=pltpu.VMEM),
        scratch_shapes=[
            pltpu.VMEM((2, RT, 128), jnp.float32),
            pltpu.VMEM((2, RT, 128), jnp.bfloat16),
            pltpu.VMEM((2, N_STAGES, RT, 128), jnp.bfloat16),
            pltpu.SemaphoreType.DMA((2, N_STAGES)),
            pltpu.SemaphoreType.DMA((2, N_STAGES)),
        ],
        compiler_params=pltpu.CompilerParams(collective_id=0),
    )(x2, Wq, Wo, K2, V2)

    return out2.reshape(1, 256, 1024)


# device time: 45647 ns/iter; 1.3654x vs baseline; 1.0628x over previous
import jax
import jax.numpy as jnp
from jax import lax
from jax.experimental import pallas as pl
from jax.experimental.pallas import tpu as pltpu

N_DEV = 16
N_STAGES = 4
SCALE = 0.08838834764831843

R = 1024
RL = 8
RT = R + RL

ORDER = ((1, 2, 4, 8), (8, 4, 2, 1))


def kernel(x, Wq, Wo, K_ext, V_ext):
    x2 = x.reshape(256, 1024)
    K2 = K_ext.reshape(4096, 256)
    V2 = V_ext.reshape(4096, 256)

    def body(x_ref, wq_ref, wo_ref, k_ref, v_ref, out_ref,
             acc_ref, send_ref, recv_ref, send_sems, recv_sems):
        my = lax.axis_index("i")

        barrier_sem = pltpu.get_barrier_semaphore()
        for d in (1, 2, 4, 8):
            pl.semaphore_signal(
                barrier_sem, inc=1,
                device_id=(my ^ d,),
                device_id_type=pl.DeviceIdType.MESH,
            )

        xq = x_ref[...].astype(jnp.bfloat16)
        wq = wq_ref[...].astype(jnp.bfloat16)
        q = jnp.dot(xq, wq, preferred_element_type=jnp.float32)
        q = (q * SCALE).astype(jnp.bfloat16)

        ri = lax.broadcasted_iota(jnp.int32, (R, 128), 0)
        ji = lax.broadcasted_iota(jnp.int32, (R, 128), 1)
        M1024 = (ri % 128 == ji).astype(jnp.float32)
        ai = lax.broadcasted_iota(jnp.int32, (RL, R), 0)
        ar = lax.broadcasted_iota(jnp.int32, (RL, R), 1)
        A8 = (ar // 128 == ai).astype(jnp.float32)

        ones_cols = jnp.ones((4096, 128), jnp.bfloat16)

        def compute_flow(g):
            kg = k_ref[:, g * 128:(g + 1) * 128].astype(jnp.bfloat16)
            vg = v_ref[:, g * 128:(g + 1) * 128].astype(jnp.bfloat16)
            vext = jnp.concatenate([vg, ones_cols], axis=1)
            q4 = jnp.concatenate(
                [q[:, (4 * g + j) * 128:(4 * g + j + 1) * 128]
                 for j in range(4)], axis=0)
            s = lax.dot_general(
                q4, kg, (((1,), (1,)), ((), ())),
                preferred_element_type=jnp.float32)
            p = jnp.exp(s)
            o_ext = jnp.dot(p.astype(jnp.bfloat16), vext,
                            preferred_element_type=jnp.float32)
            o_g = o_ext[:, :128]
            l_rep = o_ext[:, 128:]
            m_g = jnp.dot(A8, l_rep * M1024,
                          preferred_element_type=jnp.float32)
            acc_ref[g, :R, :] = o_g
            acc_ref[g, R:, :] = m_g
            send_ref[g, :R, :] = o_g.astype(jnp.bfloat16)
            send_ref[g, R:, :] = m_g.astype(jnp.bfloat16)

        def start_stage(g, stage):
            partner = my ^ ORDER[g][stage]
            rdma = pltpu.make_async_remote_copy(
                src_ref=send_ref.at[g],
                dst_ref=recv_ref.at[g, stage],
                send_sem=send_sems.at[g, stage],
                recv_sem=recv_sems.at[g, stage],
                device_id=(partner,),
                device_id_type=pl.DeviceIdType.MESH,
            )
            rdma.start()
            return rdma

        ei = lax.broadcasted_iota(jnp.int32, (R, RL), 1)
        er = lax.broadcasted_iota(jnp.int32, (R, RL), 0)
        E = (er // 128 == ei).astype(jnp.float32)

        def finalize_flow(g, a):
            l_rows = jnp.dot(E, a[R:, :],
                             preferred_element_type=jnp.float32)
            l_col = jnp.sum(l_rows * M1024, axis=1, keepdims=True)
            on = (a[:R, :] / l_col).astype(jnp.bfloat16)
            out = jnp.zeros((256, 1024), jnp.float32)
            for j in range(4):
                h = 4 * g + j
                woh = wo_ref[h * 128:(h + 1) * 128, :].astype(jnp.bfloat16)
                out = out + jnp.dot(on[j * 256:(j + 1) * 256, :], woh,
                                    preferred_element_type=jnp.float32)
            return out

        rd = {}
        compute_flow(0)
        pl.semaphore_wait(barrier_sem, N_STAGES)
        rd[(0, 0)] = start_stage(0, 0)
        compute_flow(1)
        rd[(1, 0)] = start_stage(1, 0)

        outs = []
        for stage in range(N_STAGES):
            for g in range(2):
                rd[(g, stage)].wait()
                new = acc_ref[g] + recv_ref[g, stage].astype(jnp.float32)
                if stage < N_STAGES - 1:
                    acc_ref[g] = new
                    send_ref[g] = new.astype(jnp.bfloat16)
                    rd[(g, stage + 1)] = start_stage(g, stage + 1)
                else:
                    outs.append(finalize_flow(g, new))
        out_ref[...] = outs[0] + outs[1]

    out2 = pl.pallas_call(
        body,
        out_shape=jax.ShapeDtypeStruct((256, 1024), jnp.float32),
        in_specs=[pl.BlockSpec(memory_space=pltpu.VMEM)] * 5,
        out_specs=pl.BlockSpec(memory_space=pltpu.VMEM),
        scratch_shapes=[
            pltpu.VMEM((2, RT, 128), jnp.float32),
            pltpu.VMEM((2, RT, 128), jnp.bfloat16),
            pltpu.VMEM((2, N_STAGES, RT, 128), jnp.bfloat16),
            pltpu.SemaphoreType.DMA((2, N_STAGES)),
            pltpu.SemaphoreType.DMA((2, N_STAGES)),
        ],
        compiler_params=pltpu.CompilerParams(collective_id=0),
    )(x2, Wq, Wo, K2, V2)

    return out2.reshape(1, 256, 1024)


# device time: 44680 ns/iter; 1.3949x vs baseline; 1.0216x over previous
import jax
import jax.numpy as jnp
from jax import lax
from jax.experimental import pallas as pl
from jax.experimental.pallas import tpu as pltpu

N_DEV = 16
N_STAGES = 4
N_FLOWS = 8
SCALE = 0.08838834764831843

R = 256
RL = 2
RT = R + RL

_BASE = (1, 2, 4, 8)
ORDER = tuple(tuple(_BASE[(s + r) % 4] for s in range(4)) for r in range(4))


def kernel(x, Wq, Wo, K_ext, V_ext):
    x2 = x.reshape(256, 1024)
    K2 = K_ext.reshape(4096, 256)
    V2 = V_ext.reshape(4096, 256)

    def body(x_ref, wq_ref, wo_ref, k_ref, v_ref, out_ref,
             acc_ref, send_ref, recv_ref, send_sems, recv_sems):
        my = lax.axis_index("i")

        barrier_sem = pltpu.get_barrier_semaphore()
        for d in (1, 2, 4, 8):
            pl.semaphore_signal(
                barrier_sem, inc=1,
                device_id=(my ^ d,),
                device_id_type=pl.DeviceIdType.MESH,
            )

        xq = x_ref[...].astype(jnp.bfloat16)
        wq = wq_ref[...].astype(jnp.bfloat16)
        q = jnp.dot(xq, wq, preferred_element_type=jnp.float32)
        q = (q * SCALE).astype(jnp.bfloat16)

        ri = lax.broadcasted_iota(jnp.int32, (R, 128), 0)
        ji = lax.broadcasted_iota(jnp.int32, (R, 128), 1)
        M256 = (ri % 128 == ji).astype(jnp.float32)
        ai = lax.broadcasted_iota(jnp.int32, (RL, R), 0)
        ar = lax.broadcasted_iota(jnp.int32, (RL, R), 1)
        A2 = (ar // 128 == ai).astype(jnp.float32)
        ei = lax.broadcasted_iota(jnp.int32, (R, RL), 1)
        er = lax.broadcasted_iota(jnp.int32, (R, RL), 0)
        E = (er // 128 == ei).astype(jnp.float32)

        ones_cols = jnp.ones((4096, 128), jnp.bfloat16)
        kv = {}
        for g in range(2):
            kg = k_ref[:, g * 128:(g + 1) * 128].astype(jnp.bfloat16)
            vg = v_ref[:, g * 128:(g + 1) * 128].astype(jnp.bfloat16)
            kv[g] = (kg, jnp.concatenate([vg, ones_cols], axis=1))

        def compute_flow(h):
            kg, vext = kv[h // 4]
            qh = q[:, h * 128:(h + 1) * 128]
            s = lax.dot_general(
                qh, kg, (((1,), (1,)), ((), ())),
                preferred_element_type=jnp.float32)
            p = jnp.exp(s)
            o_ext = jnp.dot(p.astype(jnp.bfloat16), vext,
                            preferred_element_type=jnp.float32)
            o_h = o_ext[:, :128]
            l_rep = o_ext[:, 128:]
            m_h = jnp.dot(A2, l_rep * M256,
                          preferred_element_type=jnp.float32)
            acc_ref[h, :R, :] = o_h
            acc_ref[h, R:, :] = m_h
            send_ref[h, :R, :] = o_h.astype(jnp.bfloat16)
            send_ref[h, R:, :] = m_h.astype(jnp.bfloat16)

        def start_stage(h, stage):
            partner = my ^ ORDER[h % 4][stage]
            rdma = pltpu.make_async_remote_copy(
                src_ref=send_ref.at[h],
                dst_ref=recv_ref.at[h, stage],
                send_sem=send_sems.at[h, stage],
                recv_sem=recv_sems.at[h, stage],
                device_id=(partner,),
                device_id_type=pl.DeviceIdType.MESH,
            )
            rdma.start()
            return rdma

        def finalize_flow(h, a):
            l_rows = jnp.dot(E, a[R:, :],
                             preferred_element_type=jnp.float32)
            l_col = jnp.sum(l_rows * M256, axis=1, keepdims=True)
            on = (a[:R, :] / l_col).astype(jnp.bfloat16)
            woh = wo_ref[h * 128:(h + 1) * 128, :].astype(jnp.bfloat16)
            return jnp.dot(on, woh, preferred_element_type=jnp.float32)

        rd = {}
        for h in range(N_FLOWS):
            compute_flow(h)
            if h == 0:
                pl.semaphore_wait(barrier_sem, N_STAGES)
            rd[(h, 0)] = start_stage(h, 0)

        out = jnp.zeros((256, 1024), jnp.float32)
        for stage in range(N_STAGES):
            for h in range(N_FLOWS):
                rd[(h, stage)].wait()
                new = acc_ref[h] + recv_ref[h, stage].astype(jnp.float32)
                if stage < N_STAGES - 1:
                    acc_ref[h] = new
                    send_ref[h] = new.astype(jnp.bfloat16)
                    rd[(h, stage + 1)] = start_stage(h, stage + 1)
                else:
                    out = out + finalize_flow(h, new)
        out_ref[...] = out

    out2 = pl.pallas_call(
        body,
        out_shape=jax.ShapeDtypeStruct((256, 1024), jnp.float32),
        in_specs=[pl.BlockSpec(memory_space=pltpu.VMEM)] * 5,
        out_specs=pl.BlockSpec(memory_space=pltpu.VMEM),
        scratch_shapes=[
            pltpu.VMEM((N_FLOWS, RT, 128), jnp.float32),
            pltpu.VMEM((N_FLOWS, RT, 128), jnp.bfloat16),
            pltpu.VMEM((N_FLOWS, N_STAGES, RT, 128), jnp.bfloat16),
            pltpu.SemaphoreType.DMA((N_FLOWS, N_STAGES)),
            pltpu.SemaphoreType.DMA((N_FLOWS, N_STAGES)),
        ],
        compiler_params=pltpu.CompilerParams(collective_id=0),
    )(x2, Wq, Wo, K2, V2)

    return out2.reshape(1, 256, 1024)
